# baseline (speedup 1.0000x reference)
import functools

import jax
import jax.numpy as jnp
from jax import lax
from jax.experimental import pallas as pl
from jax.experimental.pallas import tpu as pltpu
from jax.experimental.pallas import tpu_sc as plsc

_LANES = 16


@functools.lru_cache(maxsize=None)
def _make_sc_gather(V, D, B, NC, NS, C):
    NW = NC * NS
    assert B % NW == 0
    bpw = B // NW
    assert bpw % (2 * C) == 0
    NCH = bpw // C
    assert D % _LANES == 0

    mesh = plsc.VectorSubcoreMesh(core_axis_name="c", subcore_axis_name="s")

    @functools.partial(
        pl.kernel,
        out_type=jax.ShapeDtypeStruct((B, D), jnp.float32),
        mesh=mesh,
        scratch_types=[
            pltpu.VMEM((NCH, C), jnp.int32),
            pltpu.VMEM((C, D), jnp.float32),
            pltpu.VMEM((C, D), jnp.float32),
            pltpu.VMEM((_LANES,), jnp.int32),
            pltpu.SemaphoreType.DMA,
            pltpu.SemaphoreType.DMA,
        ],
    )
    def sc_fn(table_hbm, idx_hbm, flag_hbm, out_hbm,
              idx_v, buf0, buf1, flag_v, sem0, sem1):
        wid = lax.axis_index("s") * NC + lax.axis_index("c")
        base = wid * bpw
        pltpu.sync_copy(idx_hbm.at[wid], idx_v)
        pltpu.sync_copy(flag_hbm, flag_v)
        member = flag_v[...][0]

        @pl.when(member != 0)
        def _gather_path():
            def body(i, carry):
                g0 = 2 * i
                g1 = g0 + 1
                cp0 = pltpu.async_copy(table_hbm.at[idx_v.at[g0]], buf0, sem0)
                cp1 = pltpu.async_copy(table_hbm.at[idx_v.at[g1]], buf1, sem1)
                cp0.wait()
                pltpu.sync_copy(buf0, out_hbm.at[pl.ds(base + g0 * C, C)])
                cp1.wait()
                pltpu.sync_copy(buf1, out_hbm.at[pl.ds(base + g1 * C, C)])
                return carry

            lax.fori_loop(0, NCH // 2, body, 0)

        @pl.when(member == 0)
        def _zero_path():
            zeros = jnp.zeros((_LANES,), jnp.float32)
            for j in range(C):
                def zbody(k, carry, j=j):
                    buf0[j, pl.ds(k * _LANES, _LANES)] = zeros
                    return carry

                lax.fori_loop(0, D // _LANES, zbody, 0)

            def sbody(g, carry):
                pltpu.sync_copy(buf0, out_hbm.at[pl.ds(base + g * C, C)])
                return carry

            lax.fori_loop(0, NCH, sbody, 0)

    return sc_fn


def kernel(e_p_0, l, batch_size, task_id):
    V, P, Dm = e_p_0.shape
    B = task_id.shape[0]
    D = P * Dm
    info = plsc.get_sparse_core_info()
    NC, NS = info.num_cores, info.num_subcores
    NW = NC * NS
    C = 4

    table = e_p_0.reshape(V, D)
    idx = task_id.astype(jnp.int32).reshape(NW, (B // NW) // C, C)
    is_member = jnp.any(jnp.asarray([0, 1, 2, 3, 4], jnp.int32) == l)
    flag = jnp.where(is_member, jnp.int32(1), jnp.int32(0)) + jnp.zeros(
        (_LANES,), jnp.int32)

    out = _make_sc_gather(V, D, B, NC, NS, C)(table, idx, flag)
    return out.reshape(B, P, Dm)


# speedup vs baseline: 1.0131x; 1.0131x over previous
import functools

import jax
import jax.numpy as jnp
from jax import lax
from jax.experimental import pallas as pl
from jax.experimental.pallas import tpu as pltpu
from jax.experimental.pallas import tpu_sc as plsc

_LANES = 16
_NBUF = 4


@functools.lru_cache(maxsize=None)
def _make_sc_gather(V, D, B, NC, NS, C):
    NW = NC * NS
    assert B % NW == 0
    bpw = B // NW
    NCH = bpw // C
    assert bpw % C == 0 and NCH % _NBUF == 0 and NCH >= 2 * _NBUF
    assert D % _LANES == 0

    mesh = plsc.VectorSubcoreMesh(core_axis_name="c", subcore_axis_name="s")

    @functools.partial(
        pl.kernel,
        out_type=jax.ShapeDtypeStruct((B, D), jnp.float32),
        mesh=mesh,
        scratch_types=(
            [pltpu.VMEM((NCH, C), jnp.int32)]
            + [pltpu.VMEM((C, D), jnp.float32)] * _NBUF
            + [pltpu.VMEM((_LANES,), jnp.int32)]
            + [pltpu.SemaphoreType.DMA] * (2 * _NBUF)
        ),
    )
    def sc_fn(table_hbm, idx_hbm, flag_hbm, out_hbm, idx_v, *rest):
        bufs = rest[:_NBUF]
        flag_v = rest[_NBUF]
        gsems = rest[_NBUF + 1:2 * _NBUF + 1]
        ssems = rest[2 * _NBUF + 1:]
        wid = lax.axis_index("s") * NC + lax.axis_index("c")
        base = wid * bpw
        pltpu.sync_copy(idx_hbm.at[wid], idx_v)
        pltpu.sync_copy(flag_hbm, flag_v)
        member = flag_v[...][0]

        def g_start(g, b):
            pltpu.async_copy(table_hbm.at[idx_v.at[g]], bufs[b], gsems[b])

        def g_wait(g, b):
            pltpu.make_async_copy(
                table_hbm.at[idx_v.at[g]], bufs[b], gsems[b]).wait()

        def s_start(g, b):
            pltpu.async_copy(
                bufs[b], out_hbm.at[pl.ds(base + g * C, C)], ssems[b])

        def s_wait(g, b):
            pltpu.make_async_copy(
                bufs[b], out_hbm.at[pl.ds(base + g * C, C)], ssems[b]).wait()

        @pl.when(member != 0)
        def _gather_path():
            g_start(0, 0)
            g_start(1, 1)

            def group(i, carry):
                for b in range(_NBUF):
                    g = _NBUF * i + b
                    nb = (b + 2) % _NBUF
                    g_wait(g, b)
                    s_start(g, b)

                    @pl.when(g >= 2)
                    def _(g=g, nb=nb):
                        s_wait(g - 2, nb)

                    @pl.when(g + 2 < NCH)
                    def _(g=g, nb=nb):
                        g_start(g + 2, nb)
                return carry

            lax.fori_loop(0, NCH // _NBUF, group, 0)
            s_wait(NCH - 2, (NCH - 2) % _NBUF)
            s_wait(NCH - 1, (NCH - 1) % _NBUF)

        @pl.when(member == 0)
        def _zero_path():
            zeros = jnp.zeros((_LANES,), jnp.float32)
            buf0 = bufs[0]
            for j in range(C):
                def zbody(k, carry, j=j):
                    buf0[j, pl.ds(k * _LANES, _LANES)] = zeros
                    return carry

                lax.fori_loop(0, D // _LANES, zbody, 0)

            def sbody(g, carry):
                pltpu.sync_copy(buf0, out_hbm.at[pl.ds(base + g * C, C)])
                return carry

            lax.fori_loop(0, NCH, sbody, 0)

    return sc_fn


def kernel(e_p_0, l, batch_size, task_id):
    V, P, Dm = e_p_0.shape
    B = task_id.shape[0]
    D = P * Dm
    info = plsc.get_sparse_core_info()
    NC, NS = info.num_cores, info.num_subcores
    NW = NC * NS
    C = 1

    table = e_p_0.reshape(V, D)
    idx = task_id.astype(jnp.int32).reshape(NW, (B // NW) // C, C)
    is_member = jnp.any(jnp.asarray([0, 1, 2, 3, 4], jnp.int32) == l)
    flag = jnp.where(is_member, jnp.int32(1), jnp.int32(0)) + jnp.zeros(
        (_LANES,), jnp.int32)

    out = _make_sc_gather(V, D, B, NC, NS, C)(table, idx, flag)
    return out.reshape(B, P, Dm)
